# Initial kernel scaffold; baseline (speedup 1.0000x reference)
import functools

import jax
import jax.numpy as jnp
from jax import lax
from jax.experimental import pallas as pl
from jax.experimental.pallas import tpu as pltpu

KNB = 32
LEAK = 0.01
EPS = 1e-5


def _knn_body(p1_ref, p2_ref, idx_ref, packed_ref, *, R, N, K):
    b = pl.program_id(0)
    p1 = p1_ref[0]
    p2 = p2_ref[0]
    n2 = jnp.sum(p2 * p2, axis=0, keepdims=True)
    prod = lax.dot_general(
        p1, p2, (((0,), (0,)), ((), ())),
        preferred_element_type=jnp.float32,
        precision=lax.Precision.HIGHEST,
    )
    d = n2 - 2.0 * prod
    G = N // 128
    ik = lax.bitcast_convert_type(d, jnp.int32)
    ik = jnp.where(ik < 0, ik ^ jnp.int32(0x7FFFFFFF), ik)
    ik3 = ik.reshape(R, G, 128)
    a_io = lax.broadcasted_iota(jnp.int32, (R, G, 128), 1)
    j_io = a_io * 128 + lax.broadcasted_iota(jnp.int32, (R, G, 128), 2)
    packed_ref[...] = (ik3 & jnp.int32(-64)) | a_io
    l_io = lax.broadcasted_iota(jnp.int32, (R, 128), 1)
    for t in range(K):
        pk = packed_ref[...]
        f = jnp.min(pk, axis=1)
        m = jnp.min(f, axis=1, keepdims=True)
        lstar = jnp.min(
            jnp.where(f == m, l_io, jnp.int32(1 << 30)), axis=1, keepdims=True)
        astar = m & 63
        jstar = astar * 128 + lstar
        idx_ref[:, t:t + 1] = jstar + b * N
        packed_ref[...] = jnp.where(
            j_io == jstar[:, :, None], jnp.int32(0x7FFFFFFF), pk)


def _knn_idx_pallas(pos1, pos2, *, R=256, interpret=False):
    B, _, N = pos1.shape
    nt = N // R
    body = functools.partial(_knn_body, R=R, N=N, K=KNB)
    return pl.pallas_call(
        body,
        grid=(B, nt),
        in_specs=[
            pl.BlockSpec((1, 3, R), lambda b, i: (b, 0, i)),
            pl.BlockSpec((1, 3, N), lambda b, i: (b, 0, 0)),
        ],
        out_specs=pl.BlockSpec((R, KNB), lambda b, i: (b * (N // R) + i, 0)),
        out_shape=jax.ShapeDtypeStruct((B * N, KNB), jnp.int32),
        scratch_shapes=[pltpu.VMEM((R, N // 128, 128), jnp.int32)],
        interpret=interpret,
    )(pos1, pos2)


def kernel(feature_lst, pos_lst, cutoff, W0, W1, W2, g0, b0, g1, b1, g2, b2):
    pos1, pos2 = pos_lst[0], pos_lst[1]
    feat1, feat2 = feature_lst[0], feature_lst[1]
    B, _, N = pos1.shape

    idx_flat = _knn_idx_pallas(pos1, pos2)
    idx = idx_flat.reshape(B, N, KNB) - (jnp.arange(B, dtype=jnp.int32)[:, None, None] * N)

    def group(x, idxl):
        xt = jnp.swapaxes(x, 1, 2)
        bi = jnp.arange(xt.shape[0])[:, None, None]
        g = xt[bi, idxl]
        return jnp.transpose(g, (0, 3, 1, 2))

    def bn(x, g, b):
        mean = jnp.mean(x, axis=(0, 2, 3), keepdims=True)
        var = jnp.var(x, axis=(0, 2, 3), keepdims=True)
        xn = (x - mean) / jnp.sqrt(var + EPS)
        return xn * g[None, :, None, None] + b[None, :, None, None]

    pos2_g = group(pos2, idx)
    pos_diff = pos2_g - pos1[:, :, :, None]
    feat2_g = group(feat2, idx)
    feat1_rep = jnp.repeat(feat1[:, :, :, None], KNB, axis=3)
    x = jnp.concatenate([pos_diff, feat2_g, feat1_rep], axis=1)
    for W, g, b in zip((W0, W1, W2), (g0, g1, g2), (b0, b1, b2)):
        x = jnp.einsum('oc,bcnk->bonk', W, x)
        x = bn(x, g, b)
        x = jnp.where(x >= 0.0, x, LEAK * x)
    return jnp.max(x, axis=-1)


# speedup vs baseline: 2.6581x; 2.6581x over previous
import functools

import jax
import jax.numpy as jnp
from jax import lax
from jax.experimental import pallas as pl
from jax.experimental.pallas import tpu as pltpu

KNB = 32
LEAK = 0.01
EPS = 1e-5


def _knn_body(p1t_ref, p2_ref, idx_ref, packed_ref, *, R, N, K):
    b = pl.program_id(0)
    p1t = p1t_ref[0]
    p2 = p2_ref[0]
    n1 = jnp.sum(p1t * p1t, axis=1, keepdims=True)
    n2 = jnp.sum(p2 * p2, axis=0, keepdims=True)
    prod = lax.dot_general(
        p1t, p2, (((1,), (0,)), ((), ())),
        preferred_element_type=jnp.float32,
    )
    d = (n1 + n2) - 2.0 * prod
    G = N // 128
    ik = lax.bitcast_convert_type(d, jnp.int32)
    ik = jnp.where(ik < 0, ik ^ jnp.int32(0x7FFFFFFF), ik)
    ik3 = ik.reshape(R, G, 128)
    a_io = lax.broadcasted_iota(jnp.int32, (R, G, 128), 1)
    j_io = a_io * 128 + lax.broadcasted_iota(jnp.int32, (R, G, 128), 2)
    packed_ref[...] = (ik3 & jnp.int32(-64)) | a_io
    l_io = lax.broadcasted_iota(jnp.int32, (R, 128), 1)
    for t in range(K):
        pk = packed_ref[...]
        f = jnp.min(pk, axis=1)
        m = jnp.min(f, axis=1, keepdims=True)
        lstar = jnp.min(
            jnp.where(f == m, l_io, jnp.int32(1 << 30)), axis=1, keepdims=True)
        astar = m & 63
        jstar = astar * 128 + lstar
        idx_ref[:, t:t + 1] = jstar + b * N
        packed_ref[...] = jnp.where(
            j_io == jstar[:, :, None], jnp.int32(0x7FFFFFFF), pk)


def _knn_idx_pallas(pos1, pos2, *, R=256, interpret=False):
    B, _, N = pos1.shape
    nt = N // R
    pos1t = pos1.transpose(0, 2, 1)
    body = functools.partial(_knn_body, R=R, N=N, K=KNB)
    return pl.pallas_call(
        body,
        grid=(B, nt),
        in_specs=[
            pl.BlockSpec((1, R, 3), lambda b, i: (b, i, 0)),
            pl.BlockSpec((1, 3, N), lambda b, i: (b, 0, 0)),
        ],
        out_specs=pl.BlockSpec((R, KNB), lambda b, i: (b * (N // R) + i, 0)),
        out_shape=jax.ShapeDtypeStruct((B * N, KNB), jnp.int32),
        scratch_shapes=[pltpu.VMEM((R, N // 128, 128), jnp.int32)],
        interpret=interpret,
    )(pos1t, pos2)


def kernel(feature_lst, pos_lst, cutoff, W0, W1, W2, g0, b0, g1, b1, g2, b2):
    pos1, pos2 = pos_lst[0], pos_lst[1]
    feat1, feat2 = feature_lst[0], feature_lst[1]
    B, _, N = pos1.shape

    idx_flat = _knn_idx_pallas(pos1, pos2)
    idx = idx_flat.reshape(B, N, KNB) - (jnp.arange(B, dtype=jnp.int32)[:, None, None] * N)

    def group(x, idxl):
        xt = jnp.swapaxes(x, 1, 2)
        bi = jnp.arange(xt.shape[0])[:, None, None]
        g = xt[bi, idxl]
        return jnp.transpose(g, (0, 3, 1, 2))

    def bn(x, g, b):
        mean = jnp.mean(x, axis=(0, 2, 3), keepdims=True)
        var = jnp.var(x, axis=(0, 2, 3), keepdims=True)
        xn = (x - mean) / jnp.sqrt(var + EPS)
        return xn * g[None, :, None, None] + b[None, :, None, None]

    pos2_g = group(pos2, idx)
    pos_diff = pos2_g - pos1[:, :, :, None]
    feat2_g = group(feat2, idx)
    feat1_rep = jnp.repeat(feat1[:, :, :, None], KNB, axis=3)
    x = jnp.concatenate([pos_diff, feat2_g, feat1_rep], axis=1)
    for W, g, b in zip((W0, W1, W2), (g0, g1, g2), (b0, b1, b2)):
        x = jnp.einsum('oc,bcnk->bonk', W, x)
        x = bn(x, g, b)
        x = jnp.where(x >= 0.0, x, LEAK * x)
    return jnp.max(x, axis=-1)


# speedup vs baseline: 8.6048x; 3.2372x over previous
import functools

import jax
import jax.numpy as jnp
from jax import lax
from jax.experimental import pallas as pl
from jax.experimental.pallas import tpu as pltpu
from jax.experimental.pallas import tpu_sc as plsc

KNB = 32
CPAD = 48
LEAK = 0.01
EPS = 1e-5


def _knn_body(p1t_ref, p2_ref, idx_ref, packed_ref, *, R, N, K):
    b = pl.program_id(0)
    p1t = p1t_ref[0]
    p2 = p2_ref[0]
    n1 = jnp.sum(p1t * p1t, axis=1, keepdims=True)
    n2 = jnp.sum(p2 * p2, axis=0, keepdims=True)
    prod = lax.dot_general(
        p1t, p2, (((1,), (0,)), ((), ())),
        preferred_element_type=jnp.float32,
    )
    d = (n1 + n2) - 2.0 * prod
    G = N // 128
    ik = lax.bitcast_convert_type(d, jnp.int32)
    ik = jnp.where(ik < 0, ik ^ jnp.int32(0x7FFFFFFF), ik)
    ik3 = ik.reshape(R, G, 128)
    a_io = lax.broadcasted_iota(jnp.int32, (R, G, 128), 1)
    j_io = a_io * 128 + lax.broadcasted_iota(jnp.int32, (R, G, 128), 2)
    packed_ref[...] = (ik3 & jnp.int32(-64)) | a_io
    l_io = lax.broadcasted_iota(jnp.int32, (R, 128), 1)
    for t in range(K):
        pk = packed_ref[...]
        f = jnp.min(pk, axis=1)
        m = jnp.min(f, axis=1, keepdims=True)
        lstar = jnp.min(
            jnp.where(f == m, l_io, jnp.int32(1 << 30)), axis=1, keepdims=True)
        astar = m & 63
        jstar = astar * 128 + lstar
        idx_ref[:, t:t + 1] = jstar + b * N
        packed_ref[...] = jnp.where(
            j_io == jstar[:, :, None], jnp.int32(0x7FFFFFFF), pk)


def _knn_idx_pallas(pos1, pos2, *, R=256):
    B, _, N = pos1.shape
    nt = N // R
    pos1t = pos1.transpose(0, 2, 1)
    body = functools.partial(_knn_body, R=R, N=N, K=KNB)
    return pl.pallas_call(
        body,
        grid=(B, nt),
        in_specs=[
            pl.BlockSpec((1, R, 3), lambda b, i: (b, i, 0)),
            pl.BlockSpec((1, 3, N), lambda b, i: (b, 0, 0)),
        ],
        out_specs=pl.BlockSpec((R, KNB), lambda b, i: (b * (N // R) + i, 0)),
        out_shape=jax.ShapeDtypeStruct((B * N, KNB), jnp.int32),
        scratch_shapes=[pltpu.VMEM((R, N // 128, 128), jnp.int32)],
    )(pos1t, pos2)


def _sc_gather(table, nidx):
    P = nidx.shape[0]
    C = table.shape[1]
    info = plsc.get_sparse_core_info()
    NW = info.num_cores * info.num_subcores
    PW = P // NW
    CH = 128
    nch = PW // CH
    mesh = plsc.VectorSubcoreMesh(core_axis_name="c", subcore_axis_name="s")

    @functools.partial(
        pl.kernel, mesh=mesh,
        out_type=jax.ShapeDtypeStruct((P, C), jnp.float32),
        scratch_types=[
            pltpu.VMEM((CH,), jnp.int32),
            pltpu.VMEM((CH, C), jnp.float32),
            pltpu.SemaphoreType.DMA,
        ],
        compiler_params=pltpu.CompilerParams(use_tc_tiling_on_sc=False),
    )
    def gk(t_hbm, i_hbm, x_hbm, idx_v, buf, sem):
        cid = lax.axis_index("c")
        sid = lax.axis_index("s")
        wid = sid * info.num_cores + cid
        base = wid * PW

        def chunk(i):
            off = base + i * CH
            pltpu.sync_copy(i_hbm.at[pl.ds(off, CH)], idx_v)
            pltpu.async_copy(t_hbm.at[idx_v], buf, sem).wait()
            pltpu.sync_copy(buf, x_hbm.at[pl.ds(off, CH)])

        pl.loop(0, nch)(chunk)

    return gk(table, nidx)


def _mlp_body(xg_ref, q_ref, w0_ref, w1_ref, w2_ref, *rest, Rq, K, stage):
    ws = (w0_ref, w1_ref, w2_ref)
    params = rest[:4 * stage]
    outs = rest[4 * stage:]
    Rp = Rq * K
    x = xg_ref[...] + jnp.broadcast_to(
        q_ref[...][:, None, :], (Rq, K, CPAD)).reshape(Rp, CPAD)
    h = x
    for s in range(stage + 1):
        y = lax.dot_general(h, ws[s][...], (((1,), (1,)), ((), ())),
                            preferred_element_type=jnp.float32)
        if s == stage:
            outs[0][0] = jnp.sum(y, axis=0, keepdims=True)
            outs[1][0] = jnp.sum(y * y, axis=0, keepdims=True)
            return
        mean, sd, gam, bet = params[4 * s:4 * s + 4]
        xn = (y - mean[...]) / sd[...]
        xn = xn * gam[...] + bet[...]
        h = jnp.where(xn >= 0.0, xn, LEAK * xn)


def _mlp_final_body(xg_ref, q_ref, w0_ref, w1_ref, w2_ref, *rest, Rq, K):
    ws = (w0_ref, w1_ref, w2_ref)
    params = rest[:12]
    out_ref = rest[12]
    Rp = Rq * K
    x = xg_ref[...] + jnp.broadcast_to(
        q_ref[...][:, None, :], (Rq, K, CPAD)).reshape(Rp, CPAD)
    h = x
    for s in range(3):
        y = lax.dot_general(h, ws[s][...], (((1,), (1,)), ((), ())),
                            preferred_element_type=jnp.float32)
        mean, sd, gam, bet = params[4 * s:4 * s + 4]
        xn = (y - mean[...]) / sd[...]
        xn = xn * gam[...] + bet[...]
        h = jnp.where(xn >= 0.0, xn, LEAK * xn)
    out_ref[...] = jnp.max(h.reshape(Rq, K, h.shape[1]), axis=1)


def _stats_pass(Xg, Q1, weights, params, stage, *, Rq=256):
    P = Xg.shape[0]
    Rp = Rq * KNB
    T = P // Rp
    co = weights[stage].shape[0]
    body = functools.partial(_mlp_body, Rq=Rq, K=KNB, stage=stage)
    in_specs = [
        pl.BlockSpec((Rp, CPAD), lambda i: (i, 0)),
        pl.BlockSpec((Rq, CPAD), lambda i: (i, 0)),
    ]
    args = [Xg, Q1]
    for w in weights:
        in_specs.append(pl.BlockSpec(w.shape, lambda i: (0, 0)))
        args.append(w)
    for p in params:
        in_specs.append(pl.BlockSpec((1, p.shape[1]), lambda i: (0, 0)))
        args.append(p)
    s, ss = pl.pallas_call(
        body,
        grid=(T,),
        in_specs=in_specs,
        out_specs=(pl.BlockSpec((1, 1, co), lambda i: (i, 0, 0)),
                   pl.BlockSpec((1, 1, co), lambda i: (i, 0, 0))),
        out_shape=(jax.ShapeDtypeStruct((T, 1, co), jnp.float32),
                   jax.ShapeDtypeStruct((T, 1, co), jnp.float32)),
    )(*args)
    sm = jnp.sum(s, axis=0)
    ssm = jnp.sum(ss, axis=0)
    mean = sm / P
    var = ssm / P - mean * mean
    sd = jnp.sqrt(var + EPS)
    return mean, sd


def _final_pass(Xg, Q1, weights, params, *, Rq=256):
    P = Xg.shape[0]
    Rp = Rq * KNB
    T = P // Rp
    co = weights[2].shape[0]
    body = functools.partial(_mlp_final_body, Rq=Rq, K=KNB)
    in_specs = [
        pl.BlockSpec((Rp, CPAD), lambda i: (i, 0)),
        pl.BlockSpec((Rq, CPAD), lambda i: (i, 0)),
    ]
    args = [Xg, Q1]
    for w in weights:
        in_specs.append(pl.BlockSpec(w.shape, lambda i: (0, 0)))
        args.append(w)
    for p in params:
        in_specs.append(pl.BlockSpec((1, p.shape[1]), lambda i: (0, 0)))
        args.append(p)
    return pl.pallas_call(
        body,
        grid=(T,),
        in_specs=in_specs,
        out_specs=pl.BlockSpec((Rq, co), lambda i: (i, 0)),
        out_shape=jax.ShapeDtypeStruct((T * Rq, co), jnp.float32),
    )(*args)


def kernel(feature_lst, pos_lst, cutoff, W0, W1, W2, g0, b0, g1, b1, g2, b2):
    pos1, pos2 = pos_lst[0], pos_lst[1]
    feat1, feat2 = feature_lst[0], feature_lst[1]
    B, Cf, N = feat1.shape
    P = B * N * KNB

    nidx = _knn_idx_pallas(pos1, pos2).reshape(-1)

    p1t = pos1.transpose(0, 2, 1)
    p2t = pos2.transpose(0, 2, 1)
    f1t = feat1.transpose(0, 2, 1)
    f2t = feat2.transpose(0, 2, 1)
    zpadq = jnp.zeros((B, N, CPAD - 3 - 2 * Cf), jnp.float32)
    T2 = jnp.concatenate(
        [p2t, f2t, jnp.zeros((B, N, CPAD - 3 - Cf), jnp.float32)],
        axis=-1).reshape(B * N, CPAD)
    Q1 = jnp.concatenate(
        [-p1t, jnp.zeros((B, N, Cf), jnp.float32), f1t, zpadq],
        axis=-1).reshape(B * N, CPAD)

    Xg = _sc_gather(T2, nidx)

    W0p = jnp.concatenate(
        [W0, jnp.zeros((W0.shape[0], CPAD - W0.shape[1]), jnp.float32)],
        axis=1)
    weights = (W0p, W1, W2)
    r2 = lambda v: v.reshape(1, -1)

    mean0, sd0 = _stats_pass(Xg, Q1, weights, [], 0)
    p0 = [mean0, sd0, r2(g0), r2(b0)]
    mean1, sd1 = _stats_pass(Xg, Q1, weights, p0, 1)
    p1 = p0 + [mean1, sd1, r2(g1), r2(b1)]
    mean2, sd2 = _stats_pass(Xg, Q1, weights, p1, 2)
    p2_ = p1 + [mean2, sd2, r2(g2), r2(b2)]
    out = _final_pass(Xg, Q1, weights, p2_)

    return out.reshape(B, N, -1).transpose(0, 2, 1)


# speedup vs baseline: 18.5501x; 2.1558x over previous
import functools

import jax
import jax.numpy as jnp
from jax import lax
from jax.experimental import pallas as pl
from jax.experimental.pallas import tpu as pltpu
from jax.experimental.pallas import tpu_sc as plsc

KNB = 32
CPAD = 48
LEAK = 0.01
EPS = 1e-5


def _knn_body(p1t_ref, p2_ref, idx_ref, packed_ref, *, R, N, K):
    b = pl.program_id(0)
    p1t = p1t_ref[0]
    p2 = p2_ref[0]
    n1 = jnp.sum(p1t * p1t, axis=1, keepdims=True)
    n2 = jnp.sum(p2 * p2, axis=0, keepdims=True)
    prod = lax.dot_general(
        p1t, p2, (((1,), (0,)), ((), ())),
        preferred_element_type=jnp.float32,
    )
    d = (n1 + n2) - 2.0 * prod
    G = N // 128
    NCACHE = 6
    BIG = jnp.int32(0x7FFFFFFF)
    ik = lax.bitcast_convert_type(d, jnp.int32)
    ik = jnp.where(ik < 0, ik ^ BIG, ik)
    ik3 = ik.reshape(R, G, 128)
    a_io = lax.broadcasted_iota(jnp.int32, (R, G, 128), 1)
    l_io = lax.broadcasted_iota(jnp.int32, (R, 128), 1)
    pk0 = (ik3 & jnp.int32(-64)) | a_io

    packed_ref[...] = pk0
    M = []
    for s in range(NCACHE):
        pk = packed_ref[...]
        f = jnp.min(pk, axis=1)
        M.append(f)
        if s < NCACHE - 1:
            packed_ref[...] = jnp.where(pk == f[:, None, :], BIG, pk)

    def extract(H, cnt, t, levels):
        m = jnp.min(H, axis=1, keepdims=True)
        lstar = jnp.min(
            jnp.where(H == m, l_io, jnp.int32(1 << 30)), axis=1, keepdims=True)
        jstar = (m & 63) * 128 + lstar
        idx_ref[:, t:t + 1] = jstar + b * N
        is_l = l_io == lstar
        cnt = cnt + is_l.astype(jnp.int32)
        nh = jnp.broadcast_to(BIG, (R, 128))
        for s in range(1, len(levels)):
            nh = jnp.where(cnt == s, levels[s], nh)
        H = jnp.where(is_l, nh, H)
        return H, cnt

    H = M[0]
    cnt = jnp.zeros((R, 128), jnp.int32)
    for t in range(K):
        H, cnt = extract(H, cnt, t, M)

    viol = jnp.max(cnt)

    @pl.when(viol >= NCACHE)
    def _slow():
        j_io = a_io * 128 + lax.broadcasted_iota(jnp.int32, (R, G, 128), 2)
        packed_ref[...] = pk0
        for t in range(K):
            pk = packed_ref[...]
            f = jnp.min(pk, axis=1)
            m = jnp.min(f, axis=1, keepdims=True)
            lstar = jnp.min(
                jnp.where(f == m, l_io, jnp.int32(1 << 30)),
                axis=1, keepdims=True)
            jstar = (m & 63) * 128 + lstar
            idx_ref[:, t:t + 1] = jstar + b * N
            packed_ref[...] = jnp.where(
                j_io == jstar[:, :, None], BIG, pk)


def _knn_idx_pallas(pos1, pos2, *, R=256, interpret=False):
    B, _, N = pos1.shape
    nt = N // R
    pos1t = pos1.transpose(0, 2, 1)
    body = functools.partial(_knn_body, R=R, N=N, K=KNB)
    return pl.pallas_call(
        body,
        grid=(B, nt),
        in_specs=[
            pl.BlockSpec((1, R, 3), lambda b, i: (b, i, 0)),
            pl.BlockSpec((1, 3, N), lambda b, i: (b, 0, 0)),
        ],
        out_specs=pl.BlockSpec((R, KNB), lambda b, i: (b * (N // R) + i, 0)),
        out_shape=jax.ShapeDtypeStruct((B * N, KNB), jnp.int32),
        scratch_shapes=[pltpu.VMEM((R, N // 128, 128), jnp.int32)],
        interpret=interpret,
    )(pos1t, pos2)


def _sc_gather(table, nidx):
    P = nidx.shape[0]
    C = table.shape[1]
    info = plsc.get_sparse_core_info()
    NW = info.num_cores * info.num_subcores
    PW = P // NW
    CH = 128
    nch = PW // CH
    mesh = plsc.VectorSubcoreMesh(core_axis_name="c", subcore_axis_name="s")

    @functools.partial(
        pl.kernel, mesh=mesh,
        out_type=jax.ShapeDtypeStruct((P, C), jnp.float32),
        scratch_types=[
            pltpu.VMEM((CH,), jnp.int32),
            pltpu.VMEM((CH, C), jnp.float32),
            pltpu.SemaphoreType.DMA,
        ],
        compiler_params=pltpu.CompilerParams(use_tc_tiling_on_sc=False),
    )
    def gk(t_hbm, i_hbm, x_hbm, idx_v, buf, sem):
        cid = lax.axis_index("c")
        sid = lax.axis_index("s")
        wid = sid * info.num_cores + cid
        base = wid * PW

        def chunk(i):
            off = base + i * CH
            pltpu.sync_copy(i_hbm.at[pl.ds(off, CH)], idx_v)
            pltpu.async_copy(t_hbm.at[idx_v], buf, sem).wait()
            pltpu.sync_copy(buf, x_hbm.at[pl.ds(off, CH)])

        pl.loop(0, nch)(chunk)

    return gk(table, nidx)


def _mlp_body(xg_ref, q_ref, w0_ref, w1_ref, w2_ref, *rest, Rq, K, stage):
    ws = (w0_ref, w1_ref, w2_ref)
    params = rest[:4 * stage]
    outs = rest[4 * stage:]
    Rp = Rq * K
    x = xg_ref[...] + jnp.broadcast_to(
        q_ref[...][:, None, :], (Rq, K, CPAD)).reshape(Rp, CPAD)
    h = x
    for s in range(stage + 1):
        y = lax.dot_general(h, ws[s][...], (((1,), (1,)), ((), ())),
                            preferred_element_type=jnp.float32)
        if s == stage:
            outs[0][0] = jnp.sum(y, axis=0, keepdims=True)
            outs[1][0] = jnp.sum(y * y, axis=0, keepdims=True)
            return
        mean, sd, gam, bet = params[4 * s:4 * s + 4]
        xn = (y - mean[...]) / sd[...]
        xn = xn * gam[...] + bet[...]
        h = jnp.where(xn >= 0.0, xn, LEAK * xn)


def _mlp_final_body(xg_ref, q_ref, w0_ref, w1_ref, w2_ref, *rest, Rq, K):
    ws = (w0_ref, w1_ref, w2_ref)
    params = rest[:12]
    out_ref = rest[12]
    Rp = Rq * K
    x = xg_ref[...] + jnp.broadcast_to(
        q_ref[...][:, None, :], (Rq, K, CPAD)).reshape(Rp, CPAD)
    h = x
    for s in range(3):
        y = lax.dot_general(h, ws[s][...], (((1,), (1,)), ((), ())),
                            preferred_element_type=jnp.float32)
        mean, sd, gam, bet = params[4 * s:4 * s + 4]
        xn = (y - mean[...]) / sd[...]
        xn = xn * gam[...] + bet[...]
        h = jnp.where(xn >= 0.0, xn, LEAK * xn)
    out_ref[...] = jnp.max(h.reshape(Rq, K, h.shape[1]), axis=1)


def _stats_pass(Xg, Q1, weights, params, stage, *, Rq=256):
    P = Xg.shape[0]
    Rp = Rq * KNB
    T = P // Rp
    co = weights[stage].shape[0]
    body = functools.partial(_mlp_body, Rq=Rq, K=KNB, stage=stage)
    in_specs = [
        pl.BlockSpec((Rp, CPAD), lambda i: (i, 0)),
        pl.BlockSpec((Rq, CPAD), lambda i: (i, 0)),
    ]
    args = [Xg, Q1]
    for w in weights:
        in_specs.append(pl.BlockSpec(w.shape, lambda i: (0, 0)))
        args.append(w)
    for p in params:
        in_specs.append(pl.BlockSpec((1, p.shape[1]), lambda i: (0, 0)))
        args.append(p)
    s, ss = pl.pallas_call(
        body,
        grid=(T,),
        in_specs=in_specs,
        out_specs=(pl.BlockSpec((1, 1, co), lambda i: (i, 0, 0)),
                   pl.BlockSpec((1, 1, co), lambda i: (i, 0, 0))),
        out_shape=(jax.ShapeDtypeStruct((T, 1, co), jnp.float32),
                   jax.ShapeDtypeStruct((T, 1, co), jnp.float32)),
    )(*args)
    sm = jnp.sum(s, axis=0)
    ssm = jnp.sum(ss, axis=0)
    mean = sm / P
    var = ssm / P - mean * mean
    sd = jnp.sqrt(var + EPS)
    return mean, sd


def _final_pass(Xg, Q1, weights, params, *, Rq=256):
    P = Xg.shape[0]
    Rp = Rq * KNB
    T = P // Rp
    co = weights[2].shape[0]
    body = functools.partial(_mlp_final_body, Rq=Rq, K=KNB)
    in_specs = [
        pl.BlockSpec((Rp, CPAD), lambda i: (i, 0)),
        pl.BlockSpec((Rq, CPAD), lambda i: (i, 0)),
    ]
    args = [Xg, Q1]
    for w in weights:
        in_specs.append(pl.BlockSpec(w.shape, lambda i: (0, 0)))
        args.append(w)
    for p in params:
        in_specs.append(pl.BlockSpec((1, p.shape[1]), lambda i: (0, 0)))
        args.append(p)
    return pl.pallas_call(
        body,
        grid=(T,),
        in_specs=in_specs,
        out_specs=pl.BlockSpec((Rq, co), lambda i: (i, 0)),
        out_shape=jax.ShapeDtypeStruct((T * Rq, co), jnp.float32),
    )(*args)


def kernel(feature_lst, pos_lst, cutoff, W0, W1, W2, g0, b0, g1, b1, g2, b2):
    pos1, pos2 = pos_lst[0], pos_lst[1]
    feat1, feat2 = feature_lst[0], feature_lst[1]
    B, Cf, N = feat1.shape
    P = B * N * KNB

    nidx = _knn_idx_pallas(pos1, pos2).reshape(-1)

    p1t = pos1.transpose(0, 2, 1)
    p2t = pos2.transpose(0, 2, 1)
    f1t = feat1.transpose(0, 2, 1)
    f2t = feat2.transpose(0, 2, 1)
    zpadq = jnp.zeros((B, N, CPAD - 3 - 2 * Cf), jnp.float32)
    T2 = jnp.concatenate(
        [p2t, f2t, jnp.zeros((B, N, CPAD - 3 - Cf), jnp.float32)],
        axis=-1).reshape(B * N, CPAD)
    Q1 = jnp.concatenate(
        [-p1t, jnp.zeros((B, N, Cf), jnp.float32), f1t, zpadq],
        axis=-1).reshape(B * N, CPAD)

    Xg = _sc_gather(T2, nidx)

    W0p = jnp.concatenate(
        [W0, jnp.zeros((W0.shape[0], CPAD - W0.shape[1]), jnp.float32)],
        axis=1)
    weights = (W0p, W1, W2)
    r2 = lambda v: v.reshape(1, -1)

    mean0, sd0 = _stats_pass(Xg, Q1, weights, [], 0)
    p0 = [mean0, sd0, r2(g0), r2(b0)]
    mean1, sd1 = _stats_pass(Xg, Q1, weights, p0, 1)
    p1 = p0 + [mean1, sd1, r2(g1), r2(b1)]
    mean2, sd2 = _stats_pass(Xg, Q1, weights, p1, 2)
    p2_ = p1 + [mean2, sd2, r2(g2), r2(b2)]
    out = _final_pass(Xg, Q1, weights, p2_)

    return out.reshape(B, N, -1).transpose(0, 2, 1)


# speedup vs baseline: 19.5327x; 1.0530x over previous
import functools

import jax
import jax.numpy as jnp
from jax import lax
from jax.experimental import pallas as pl
from jax.experimental.pallas import tpu as pltpu
from jax.experimental.pallas import tpu_sc as plsc

KNB = 32
CPAD = 48
LEAK = 0.01
EPS = 1e-5


def _knn_body(p1t_ref, p2_ref, idx_ref, packed_ref, *, R, N, K):
    b = pl.program_id(0)
    p1t = p1t_ref[0]
    p2 = p2_ref[0]
    n1 = jnp.sum(p1t * p1t, axis=1, keepdims=True)
    n2 = jnp.sum(p2 * p2, axis=0, keepdims=True)
    prod = lax.dot_general(
        p1t, p2, (((1,), (0,)), ((), ())),
        preferred_element_type=jnp.float32,
    )
    d = (n1 + n2) - 2.0 * prod
    G = N // 128
    NCACHE = 6
    BIG = jnp.int32(0x7FFFFFFF)
    ik = lax.bitcast_convert_type(d, jnp.int32)
    ik = jnp.where(ik < 0, ik ^ BIG, ik)
    ik3 = ik.reshape(R, G, 128)
    a_io = lax.broadcasted_iota(jnp.int32, (R, G, 128), 1)
    l_io = lax.broadcasted_iota(jnp.int32, (R, 128), 1)
    pk0 = (ik3 & jnp.int32(-64)) | a_io

    pk = pk0
    M = [jnp.min(pk, axis=1)]
    for s in range(1, NCACHE):
        pk = jnp.where(pk == M[-1][:, None, :], BIG, pk)
        M.append(jnp.min(pk, axis=1))

    def extract(H, cnt, t, levels):
        m = jnp.min(H, axis=1, keepdims=True)
        lstar = jnp.min(
            jnp.where(H == m, l_io, jnp.int32(1 << 30)), axis=1, keepdims=True)
        jstar = (m & 63) * 128 + lstar
        idx_ref[:, t:t + 1] = jstar + b * N
        is_l = l_io == lstar
        cnt = cnt + is_l.astype(jnp.int32)
        nh = jnp.broadcast_to(BIG, (R, 128))
        for s in range(1, len(levels)):
            nh = jnp.where(cnt == s, levels[s], nh)
        H = jnp.where(is_l, nh, H)
        return H, cnt

    H = M[0]
    cnt = jnp.zeros((R, 128), jnp.int32)
    for t in range(K):
        H, cnt = extract(H, cnt, t, M)

    viol = jnp.max(cnt)

    @pl.when(viol >= NCACHE)
    def _slow():
        j_io = a_io * 128 + lax.broadcasted_iota(jnp.int32, (R, G, 128), 2)
        packed_ref[...] = pk0
        for t in range(K):
            pk = packed_ref[...]
            f = jnp.min(pk, axis=1)
            m = jnp.min(f, axis=1, keepdims=True)
            lstar = jnp.min(
                jnp.where(f == m, l_io, jnp.int32(1 << 30)),
                axis=1, keepdims=True)
            jstar = (m & 63) * 128 + lstar
            idx_ref[:, t:t + 1] = jstar + b * N
            packed_ref[...] = jnp.where(
                j_io == jstar[:, :, None], BIG, pk)


def _knn_idx_pallas(pos1, pos2, *, R=256, interpret=False):
    B, _, N = pos1.shape
    nt = N // R
    pos1t = pos1.transpose(0, 2, 1)
    body = functools.partial(_knn_body, R=R, N=N, K=KNB)
    return pl.pallas_call(
        body,
        grid=(B, nt),
        in_specs=[
            pl.BlockSpec((1, R, 3), lambda b, i: (b, i, 0)),
            pl.BlockSpec((1, 3, N), lambda b, i: (b, 0, 0)),
        ],
        out_specs=pl.BlockSpec((R, KNB), lambda b, i: (b * (N // R) + i, 0)),
        out_shape=jax.ShapeDtypeStruct((B * N, KNB), jnp.int32),
        scratch_shapes=[pltpu.VMEM((R, N // 128, 128), jnp.int32)],
        interpret=interpret,
    )(pos1t, pos2)


def _sc_gather(table, nidx):
    P = nidx.shape[0]
    C = table.shape[1]
    info = plsc.get_sparse_core_info()
    NW = info.num_cores * info.num_subcores
    PW = P // NW
    CH = 128
    SUB = 4
    SC_ROWS = CH * SUB
    NBUF = 2
    nsc = PW // SC_ROWS
    mesh = plsc.VectorSubcoreMesh(core_axis_name="c", subcore_axis_name="s")
    idx2 = nidx.reshape(P // CH, CH)
    assert nsc % NBUF == 0

    @functools.partial(
        pl.kernel, mesh=mesh,
        out_type=jax.ShapeDtypeStruct((P, C), jnp.float32),
        scratch_types=[
            pltpu.VMEM((SUB, CH), jnp.int32),
            pltpu.VMEM((SUB, CH), jnp.int32),
            pltpu.VMEM((SC_ROWS, C), jnp.float32),
            pltpu.VMEM((SC_ROWS, C), jnp.float32),
            pltpu.SemaphoreType.DMA,
            pltpu.SemaphoreType.DMA,
            pltpu.SemaphoreType.DMA,
            pltpu.SemaphoreType.DMA,
            pltpu.SemaphoreType.DMA,
            pltpu.SemaphoreType.DMA,
        ],
        compiler_params=pltpu.CompilerParams(use_tc_tiling_on_sc=False),
    )
    def gk(t_hbm, i_hbm, x_hbm, v0, v1, buf0, buf1,
           is0, is1, gs0, gs1, os0, os1):
        cid = lax.axis_index("c")
        sid = lax.axis_index("s")
        wid = sid * info.num_cores + cid
        base = wid * nsc

        def idx_copy(i, v, sem):
            return pltpu.make_async_copy(
                i_hbm.at[pl.ds((base + i) * SUB, SUB)], v, sem)

        def gather_copy(v, buf, sem, j):
            return pltpu.make_async_copy(
                t_hbm.at[v.at[j]], buf.at[pl.ds(j * CH, CH)], sem)

        def start_gather(v, buf, sem):
            for j in range(SUB):
                gather_copy(v, buf, sem, j).start()

        def wait_gather(v, buf, sem):
            for j in range(SUB):
                gather_copy(v, buf, sem, j).wait()

        def out_copy(i, buf, sem):
            return pltpu.make_async_copy(
                buf, x_hbm.at[pl.ds((base + i) * SC_ROWS, SC_ROWS)], sem)

        idx_copy(0, v0, is0).start()
        idx_copy(0, v0, is0).wait()
        start_gather(v0, buf0, gs0)
        idx_copy(1, v1, is1).start()

        def body(g):
            i0 = 2 * g
            i1 = 2 * g + 1
            wait_gather(v0, buf0, gs0)
            idx_copy(i1, v1, is1).wait()

            @pl.when(g >= 1)
            def _():
                out_copy(i1 - 2, buf1, os1).wait()

            start_gather(v1, buf1, gs1)
            out_copy(i0, buf0, os0).start()

            @pl.when(i0 + 2 < nsc)
            def _():
                idx_copy(i0 + 2, v0, is0).start()

            wait_gather(v1, buf1, gs1)
            out_copy(i1, buf1, os1).start()

            @pl.when(i1 + 2 < nsc)
            def _():
                idx_copy(i1 + 2, v1, is1).start()

            @pl.when(i0 + 2 < nsc)
            def _():
                idx_copy(i0 + 2, v0, is0).wait()
                out_copy(i0, buf0, os0).wait()
                start_gather(v0, buf0, gs0)

        pl.loop(0, nsc // 2)(body)
        out_copy(nsc - 2, buf0, os0).wait()
        out_copy(nsc - 1, buf1, os1).wait()

    return gk(table, idx2)


def _mlp_body(xg_ref, q_ref, w0_ref, w1_ref, w2_ref, *rest, Rq, K, stage):
    ws = (w0_ref, w1_ref, w2_ref)
    params = rest[:4 * stage]
    outs = rest[4 * stage:]
    Rp = Rq * K
    x = xg_ref[...] + jnp.broadcast_to(
        q_ref[...][:, None, :], (Rq, K, CPAD)).reshape(Rp, CPAD)
    h = x
    for s in range(stage + 1):
        y = lax.dot_general(h, ws[s][...], (((1,), (1,)), ((), ())),
                            preferred_element_type=jnp.float32)
        if s == stage:
            outs[0][0] = jnp.sum(y, axis=0, keepdims=True)
            outs[1][0] = jnp.sum(y * y, axis=0, keepdims=True)
            return
        mean, sd, gam, bet = params[4 * s:4 * s + 4]
        xn = (y - mean[...]) / sd[...]
        xn = xn * gam[...] + bet[...]
        h = jnp.where(xn >= 0.0, xn, LEAK * xn)


def _mlp_final_body(xg_ref, q_ref, w0_ref, w1_ref, w2_ref, *rest, Rq, K):
    ws = (w0_ref, w1_ref, w2_ref)
    params = rest[:12]
    out_ref = rest[12]
    Rp = Rq * K
    x = xg_ref[...] + jnp.broadcast_to(
        q_ref[...][:, None, :], (Rq, K, CPAD)).reshape(Rp, CPAD)
    h = x
    for s in range(3):
        y = lax.dot_general(h, ws[s][...], (((1,), (1,)), ((), ())),
                            preferred_element_type=jnp.float32)
        mean, sd, gam, bet = params[4 * s:4 * s + 4]
        xn = (y - mean[...]) / sd[...]
        xn = xn * gam[...] + bet[...]
        h = jnp.where(xn >= 0.0, xn, LEAK * xn)
    out_ref[...] = jnp.max(h.reshape(Rq, K, h.shape[1]), axis=1)


def _stats_pass(Xg, Q1, weights, params, stage, *, Rq=256):
    P = Xg.shape[0]
    Rp = Rq * KNB
    T = P // Rp
    co = weights[stage].shape[0]
    body = functools.partial(_mlp_body, Rq=Rq, K=KNB, stage=stage)
    in_specs = [
        pl.BlockSpec((Rp, CPAD), lambda i: (i, 0)),
        pl.BlockSpec((Rq, CPAD), lambda i: (i, 0)),
    ]
    args = [Xg, Q1]
    for w in weights:
        in_specs.append(pl.BlockSpec(w.shape, lambda i: (0, 0)))
        args.append(w)
    for p in params:
        in_specs.append(pl.BlockSpec((1, p.shape[1]), lambda i: (0, 0)))
        args.append(p)
    s, ss = pl.pallas_call(
        body,
        grid=(T,),
        in_specs=in_specs,
        out_specs=(pl.BlockSpec((1, 1, co), lambda i: (i, 0, 0)),
                   pl.BlockSpec((1, 1, co), lambda i: (i, 0, 0))),
        out_shape=(jax.ShapeDtypeStruct((T, 1, co), jnp.float32),
                   jax.ShapeDtypeStruct((T, 1, co), jnp.float32)),
    )(*args)
    sm = jnp.sum(s, axis=0)
    ssm = jnp.sum(ss, axis=0)
    mean = sm / P
    var = ssm / P - mean * mean
    sd = jnp.sqrt(var + EPS)
    return mean, sd


def _final_pass(Xg, Q1, weights, params, *, Rq=256):
    P = Xg.shape[0]
    Rp = Rq * KNB
    T = P // Rp
    co = weights[2].shape[0]
    body = functools.partial(_mlp_final_body, Rq=Rq, K=KNB)
    in_specs = [
        pl.BlockSpec((Rp, CPAD), lambda i: (i, 0)),
        pl.BlockSpec((Rq, CPAD), lambda i: (i, 0)),
    ]
    args = [Xg, Q1]
    for w in weights:
        in_specs.append(pl.BlockSpec(w.shape, lambda i: (0, 0)))
        args.append(w)
    for p in params:
        in_specs.append(pl.BlockSpec((1, p.shape[1]), lambda i: (0, 0)))
        args.append(p)
    return pl.pallas_call(
        body,
        grid=(T,),
        in_specs=in_specs,
        out_specs=pl.BlockSpec((Rq, co), lambda i: (i, 0)),
        out_shape=jax.ShapeDtypeStruct((T * Rq, co), jnp.float32),
    )(*args)


def kernel(feature_lst, pos_lst, cutoff, W0, W1, W2, g0, b0, g1, b1, g2, b2):
    pos1, pos2 = pos_lst[0], pos_lst[1]
    feat1, feat2 = feature_lst[0], feature_lst[1]
    B, Cf, N = feat1.shape
    P = B * N * KNB

    nidx = _knn_idx_pallas(pos1, pos2).reshape(-1)

    p1t = pos1.transpose(0, 2, 1)
    p2t = pos2.transpose(0, 2, 1)
    f1t = feat1.transpose(0, 2, 1)
    f2t = feat2.transpose(0, 2, 1)
    zpadq = jnp.zeros((B, N, CPAD - 3 - 2 * Cf), jnp.float32)
    T2 = jnp.concatenate(
        [p2t, f2t, jnp.zeros((B, N, CPAD - 3 - Cf), jnp.float32)],
        axis=-1).reshape(B * N, CPAD)
    Q1 = jnp.concatenate(
        [-p1t, jnp.zeros((B, N, Cf), jnp.float32), f1t, zpadq],
        axis=-1).reshape(B * N, CPAD)

    Xg = _sc_gather(T2, nidx)

    W0p = jnp.concatenate(
        [W0, jnp.zeros((W0.shape[0], CPAD - W0.shape[1]), jnp.float32)],
        axis=1)
    weights = (W0p, W1, W2)
    r2 = lambda v: v.reshape(1, -1)

    mean0, sd0 = _stats_pass(Xg, Q1, weights, [], 0)
    p0 = [mean0, sd0, r2(g0), r2(b0)]
    mean1, sd1 = _stats_pass(Xg, Q1, weights, p0, 1)
    p1 = p0 + [mean1, sd1, r2(g1), r2(b1)]
    mean2, sd2 = _stats_pass(Xg, Q1, weights, p1, 2)
    p2_ = p1 + [mean2, sd2, r2(g2), r2(b2)]
    out = _final_pass(Xg, Q1, weights, p2_)

    return out.reshape(B, N, -1).transpose(0, 2, 1)
